# baseline (speedup 1.0000x reference)
import functools

import jax
import jax.numpy as jnp
from jax import lax
from jax.experimental import pallas as pl
from jax.experimental.pallas import tpu as pltpu
from jax.experimental.pallas import tpu_sc as plsc

HIDDEN = 128
EPS = 1e-12
NC = 2
NS = 16
NW = NC * NS

CH = 128
NCHUNK = 128
NBUF = 2
NG = CH // 16


def _rsqrt_newton_v(x):
    i = lax.bitcast_convert_type(x, jnp.int32)
    i = jnp.full((16,), 0x5F3759DF, jnp.int32) - lax.shift_right_arithmetic(i, 1)
    y = lax.bitcast_convert_type(i, jnp.float32)
    for _ in range(4):
        y = y * (1.5 - 0.5 * x * y * y)
    return y


def fused_embed_ln(ids_flat, tt_flat, word_emb, pos_emb, type_emb, gamma, beta):
    n = ids_flat.shape[0]
    mesh = plsc.VectorSubcoreMesh(core_axis_name="c", subcore_axis_name="s")

    scratch = (
        [pltpu.VMEM((CH, HIDDEN), jnp.float32) for _ in range(NBUF)]
        + [pltpu.VMEM((CH,), jnp.int32) for _ in range(NBUF)]
        + [pltpu.VMEM((CH,), jnp.int32) for _ in range(NBUF)]
        + [pltpu.VMEM((CH, HIDDEN), jnp.float32)]
        + [pltpu.VMEM((HIDDEN, CH), jnp.float32)]
        + [pltpu.VMEM((HIDDEN, 16), jnp.float32)]
        + [pltpu.VMEM((HIDDEN, 16), jnp.float32)]
        + [pltpu.VMEM((HIDDEN, 16), jnp.float32)]
        + [pltpu.VMEM((2, HIDDEN), jnp.float32)]
        + [pltpu.VMEM((HIDDEN,), jnp.float32)]
        + [pltpu.VMEM((HIDDEN,), jnp.float32)]
        + [pltpu.SemaphoreType.DMA for _ in range(3 * NBUF)]
    )

    @functools.partial(
        pl.kernel,
        out_type=jax.ShapeDtypeStruct((n, HIDDEN), jnp.float32),
        mesh=mesh,
        scratch_types=scratch,
        compiler_params=pltpu.CompilerParams(needs_layout_passes=False),
    )
    def k(ids_hbm, tt_hbm, table_hbm, pos_hbm, type_hbm, gam_hbm, bet_hbm,
          out_hbm, *scr):
        rows = scr[0:NBUF]
        idx = scr[NBUF:2 * NBUF]
        ttv = scr[2 * NBUF:3 * NBUF]
        pos_v, post0, dtx, gx, bx, tv, gv, bv = scr[3 * NBUF:3 * NBUF + 8]
        sem_i = scr[3 * NBUF + 8:3 * NBUF + 8 + NBUF]
        sem_g = scr[3 * NBUF + 8 + NBUF:3 * NBUF + 8 + 2 * NBUF]
        sem_w = scr[3 * NBUF + 8 + 2 * NBUF:]

        wid = lax.axis_index("s") * NC + lax.axis_index("c")
        b0 = (wid // 4) * NCHUNK
        p0 = (wid % 4) * CH

        pltpu.sync_copy(pos_hbm.at[pl.ds(p0, CH)], pos_v)
        pltpu.sync_copy(type_hbm, tv)
        pltpu.sync_copy(gam_hbm, gv)
        pltpu.sync_copy(bet_hbm, bv)

        iota = lax.iota(jnp.int32, 16)

        def build_f(f, carry):
            ff = jnp.full((16,), f, jnp.int32)
            z = jnp.full((16,), 0, jnp.int32)
            o = jnp.full((16,), 1, jnp.int32)
            t0s = plsc.load_gather(tv, [z, ff])
            t1s = plsc.load_gather(tv, [o, ff])
            dtx[f, :] = t1s - t0s
            gx[f, :] = plsc.load_gather(gv, [ff])
            bx[f, :] = plsc.load_gather(bv, [ff])

            def build_g(g, c2):
                tok = g * 16 + iota
                post0[f, pl.ds(g * 16, 16)] = (
                    plsc.load_gather(pos_v, [tok, ff]) + t0s)
                return c2

            lax.fori_loop(0, NG, build_g, 0)
            return carry

        lax.fori_loop(0, HIDDEN, build_f, 0)

        def row_base(cc):
            return (b0 + cc) * 512 + p0

        def ids_descs(cc, b):
            rb = row_base(cc)
            return (pltpu.make_async_copy(ids_hbm.at[pl.ds(rb, CH)], idx[b], sem_i[b]),
                    pltpu.make_async_copy(tt_hbm.at[pl.ds(rb, CH)], ttv[b], sem_i[b]))

        def gather_desc(b):
            return pltpu.make_async_copy(table_hbm.at[idx[b]], rows[b], sem_g[b])

        def write_desc(cc, b):
            return pltpu.make_async_copy(rows[b], out_hbm.at[pl.ds(row_base(cc), CH)],
                                         sem_w[b])

        def start_ids(cc, b):
            d1, d2 = ids_descs(cc, b)
            d1.start()
            d2.start()

        def wait_ids(cc, b):
            d1, d2 = ids_descs(cc, b)
            d1.wait()
            d2.wait()

        def compute_chunk(b):
            rows_b = rows[b]
            ttv_b = ttv[b]
            ttfs = [ttv_b[pl.ds(g * 16, 16)].astype(jnp.float32)
                    for g in range(NG)]
            zeros = jnp.full((16,), 0.0, jnp.float32)

            def f_body(f, carry):
                ss, qq = carry
                ff = jnp.full((16,), f, jnp.int32)
                dtv = dtx[f, :]
                nss = []
                nqq = []
                for g in range(NG):
                    tok = g * 16 + iota
                    w = plsc.load_gather(rows_b, [tok, ff])
                    p = post0[f, pl.ds(g * 16, 16)]
                    x = (w + p) + ttfs[g] * dtv
                    plsc.store_scatter(rows_b, [tok, ff], x)
                    nss.append(ss[g] + x)
                    nqq.append(qq[g] + x * x)
                return (tuple(nss), tuple(nqq))

            ss, qq = lax.fori_loop(
                0, HIDDEN, f_body,
                (tuple([zeros] * NG), tuple([zeros] * NG)))

            mus = []
            rss = []
            for g in range(NG):
                mu = ss[g] * (1.0 / HIDDEN)
                var = qq[g] * (1.0 / HIDDEN) - mu * mu
                mus.append(mu)
                rss.append(_rsqrt_newton_v(var + EPS))

            def f2_body(f, carry):
                ff = jnp.full((16,), f, jnp.int32)
                gvv = gx[f, :]
                bvv = bx[f, :]
                for g in range(NG):
                    tok = g * 16 + iota
                    x = plsc.load_gather(rows_b, [tok, ff])
                    y = ((x - mus[g]) * rss[g]) * gvv + bvv
                    plsc.store_scatter(rows_b, [tok, ff], y)
                return carry

            lax.fori_loop(0, HIDDEN, f2_body, 0)

        start_ids(0, 0)
        start_ids(1, 1)
        wait_ids(0, 0)
        gather_desc(0).start()

        def outer(i, carry):
            c2 = i * NBUF
            for boff in range(NBUF):
                c = c2 + boff
                nb1 = (boff + 1) % NBUF

                @pl.when(c + 1 < NCHUNK)
                def _():
                    wait_ids(c + 1, nb1)

                    @pl.when(c + 1 >= NBUF)
                    def _():
                        write_desc(c + 1 - NBUF, nb1).wait()

                    gather_desc(nb1).start()

                gather_desc(boff).wait()
                compute_chunk(boff)

                @pl.when(c + 2 < NCHUNK)
                def _():
                    start_ids(c + 2, boff)

                write_desc(c, boff).start()
            return carry

        lax.fori_loop(0, NCHUNK // NBUF, outer, 0)

        for kk in range(NBUF):
            write_desc(NCHUNK - NBUF + kk, kk).wait()

    return k(ids_flat, tt_flat, word_emb, pos_emb, type_emb, gamma, beta)


def kernel(input_ids, token_type_ids, word_emb, pos_emb, type_emb, ln_gamma, ln_beta):
    b, s = input_ids.shape
    n = b * s
    ids_flat = input_ids.reshape(n).astype(jnp.int32)
    tt_flat = token_type_ids.reshape(n).astype(jnp.int32)
    out = fused_embed_ln(ids_flat, tt_flat, word_emb, pos_emb, type_emb,
                         ln_gamma, ln_beta)
    return out.reshape(b, s, HIDDEN)


# speedup vs baseline: 14.0640x; 14.0640x over previous
import functools

import jax
import jax.numpy as jnp
from jax import lax
from jax.experimental import pallas as pl
from jax.experimental.pallas import tpu as pltpu
from jax.experimental.pallas import tpu_sc as plsc

HIDDEN = 128
EPS = 1e-12

NC = 2
NS = 16
NW = NC * NS


def _sc_gather(ids_flat, table, ch):
    n = ids_flat.shape[0]
    width = table.shape[1]
    dt = table.dtype
    per_w = n // NW
    steps = per_w // ch
    mesh = plsc.VectorSubcoreMesh(core_axis_name="c", subcore_axis_name="s")

    @functools.partial(
        pl.kernel,
        out_type=jax.ShapeDtypeStruct((n, width), dt),
        mesh=mesh,
        scratch_types=[
            pltpu.VMEM((ch,), jnp.int32),
            pltpu.VMEM((ch, width), dt),
            pltpu.SemaphoreType.DMA,
        ],
    )
    def gather_k(ids_hbm, table_hbm, out_hbm, idx_v, rows_v, sem):
        wid = lax.axis_index("s") * NC + lax.axis_index("c")

        def body(g, carry):
            base = wid * per_w + g * ch
            pltpu.sync_copy(ids_hbm.at[pl.ds(base, ch)], idx_v)
            pltpu.async_copy(table_hbm.at[idx_v], rows_v, sem).wait()
            pltpu.sync_copy(rows_v, out_hbm.at[pl.ds(base, ch)])
            return carry

        lax.fori_loop(0, steps, body, 0)

    return gather_k(ids_flat, table)


def _tc_ln_body(wg_ref, tt_ref, pos_ref, type_ref, gam_ref, bet_ref, out_ref):
    tt = tt_ref[...].astype(jnp.float32)[:, :, None]
    t0 = type_ref[0:1, :]
    dt = (type_ref[1:2, :] - t0)[None, :, :]
    wg = wg_ref[...].astype(jnp.float32)
    x = wg + pos_ref[...][None, :, :] + (t0[None, :, :] + tt * dt)
    mu = jnp.mean(x, axis=-1, keepdims=True)
    xc = x - mu
    var = jnp.mean(xc * xc, axis=-1, keepdims=True)
    y = xc * lax.rsqrt(var + EPS)
    out_ref[...] = y * gam_ref[...][None, :, :] + bet_ref[...][None, :, :]


def _tc_ln_body_acc(acc_ref, wg_ref, tt_ref, pos_ref, type_ref, gam_ref, bet_ref,
                    out_ref):
    del acc_ref
    _tc_ln_body(wg_ref, tt_ref, pos_ref, type_ref, gam_ref, bet_ref, out_ref)


def kernel(input_ids, token_type_ids, word_emb, pos_emb, type_emb, ln_gamma, ln_beta):
    b, s = input_ids.shape
    n = b * s
    ids_flat = input_ids.reshape(n).astype(jnp.int32)
    tt2 = token_type_ids.astype(jnp.int32)
    g2 = ln_gamma.reshape(1, HIDDEN)
    b2 = ln_beta.reshape(1, HIDDEN)

    K = 8
    bc = b // K
    nk = bc * s
    nb = 32
    nblk = bc // nb

    out = None
    for k in range(K):
        wg_k = _sc_gather(ids_flat[k * nk:(k + 1) * nk], word_emb, ch=512)
        wg_k = wg_k.reshape(bc, s, HIDDEN)
        tt_k = tt2[k * bc:(k + 1) * bc]
        out_spec = pl.BlockSpec(
            (nb, s, HIDDEN), lambda i, _k=k: (_k * nblk + i, 0, 0))
        data_specs = [
            pl.BlockSpec((nb, s, HIDDEN), lambda i: (i, 0, 0)),
            pl.BlockSpec((nb, s), lambda i: (i, 0)),
            pl.BlockSpec((s, HIDDEN), lambda i: (0, 0)),
            pl.BlockSpec((2, HIDDEN), lambda i: (0, 0)),
            pl.BlockSpec((1, HIDDEN), lambda i: (0, 0)),
            pl.BlockSpec((1, HIDDEN), lambda i: (0, 0)),
        ]
        if k == 0:
            out = pl.pallas_call(
                _tc_ln_body,
                grid=(nblk,),
                in_specs=data_specs,
                out_specs=out_spec,
                out_shape=jax.ShapeDtypeStruct((b, s, HIDDEN), jnp.float32),
            )(wg_k, tt_k, pos_emb, type_emb, g2, b2)
        else:
            out = pl.pallas_call(
                _tc_ln_body_acc,
                grid=(nblk,),
                in_specs=[pl.BlockSpec(memory_space=pltpu.MemorySpace.HBM)]
                + data_specs,
                out_specs=out_spec,
                out_shape=jax.ShapeDtypeStruct((b, s, HIDDEN), jnp.float32),
                input_output_aliases={0: 0},
            )(out, wg_k, tt_k, pos_emb, type_emb, g2, b2)
    return out
